# baseline (speedup 1.0000x reference)
import functools

import jax
import jax.numpy as jnp
from jax import lax
from jax.experimental import pallas as pl
from jax.experimental.pallas import tpu as pltpu
from jax.experimental.pallas import tpu_sc as plsc

_NUM_TOKENS = 100
_D_MODEL = 4096
_BATCH = 4
_NUM_CORES = 2

_mesh = plsc.ScalarSubcoreMesh(axis_name="c", num_cores=_NUM_CORES)


@functools.partial(
    pl.kernel,
    mesh=_mesh,
    out_type=jax.ShapeDtypeStruct((_BATCH, _NUM_TOKENS, _D_MODEL), jnp.float32),
    scratch_types=[pltpu.SemaphoreType.DMA],
)
def _broadcast_kernel(table_hbm, out_hbm, sem):
    core = lax.axis_index("c")
    copies = [
        pltpu.async_copy(table_hbm, out_hbm.at[core * 2 + i], sem)
        for i in range(_BATCH // _NUM_CORES)
    ]
    for c in copies:
        c.wait()


def kernel(batch_size, prompt_embeddings):
    del batch_size
    return _broadcast_kernel(prompt_embeddings)


# speedup vs baseline: 7.0608x; 7.0608x over previous
import functools

import jax
import jax.numpy as jnp
from jax import lax
from jax.experimental import pallas as pl
from jax.experimental.pallas import tpu as pltpu
from jax.experimental.pallas import tpu_sc as plsc

_NUM_TOKENS = 100
_D_MODEL = 4096
_BATCH = 4
_NUM_CORES = 2
_NUM_SUBCORES = 16
_NUM_WORKERS = _NUM_CORES * _NUM_SUBCORES
_STRIPE = _D_MODEL // _NUM_WORKERS
_ROWS_A = 48
_ROWS_B = _NUM_TOKENS - _ROWS_A

_mesh = plsc.VectorSubcoreMesh(core_axis_name="c", subcore_axis_name="s")


@functools.partial(
    pl.kernel,
    mesh=_mesh,
    out_type=jax.ShapeDtypeStruct((_BATCH, _NUM_TOKENS, _D_MODEL), jnp.float32),
    scratch_types=[
        pltpu.VMEM((_ROWS_A, _STRIPE), jnp.float32),
        pltpu.VMEM((_ROWS_B, _STRIPE), jnp.float32),
        pltpu.SemaphoreType.DMA,
    ],
)
def _broadcast_kernel(table_hbm, out_hbm, buf_a, buf_b, sem):
    wid = lax.axis_index("s") * _NUM_CORES + lax.axis_index("c")
    col = wid * _STRIPE
    pltpu.sync_copy(table_hbm.at[pl.ds(0, _ROWS_A), pl.ds(col, _STRIPE)], buf_a)
    copies = [
        pltpu.async_copy(
            buf_a, out_hbm.at[b].at[pl.ds(0, _ROWS_A), pl.ds(col, _STRIPE)], sem
        )
        for b in range(_BATCH)
    ]
    pltpu.sync_copy(
        table_hbm.at[pl.ds(_ROWS_A, _ROWS_B), pl.ds(col, _STRIPE)], buf_b
    )
    copies += [
        pltpu.async_copy(
            buf_b, out_hbm.at[b].at[pl.ds(_ROWS_A, _ROWS_B), pl.ds(col, _STRIPE)], sem
        )
        for b in range(_BATCH)
    ]
    for c in copies:
        c.wait()


def kernel(batch_size, prompt_embeddings):
    del batch_size
    return _broadcast_kernel(prompt_embeddings)


# speedup vs baseline: 7.2207x; 1.0227x over previous
import functools

import jax
import jax.numpy as jnp
from jax.experimental import pallas as pl
from jax.experimental.pallas import tpu as pltpu

_NUM_TOKENS = 100
_D_MODEL = 4096
_BATCH = 4
_ROW_BLOCK = 8
_NUM_ROW_BLOCKS = (_NUM_TOKENS + _ROW_BLOCK - 1) // _ROW_BLOCK


def _body(table_ref, out_ref):
    out_ref[0] = table_ref[...]


def kernel(batch_size, prompt_embeddings):
    del batch_size
    return pl.pallas_call(
        _body,
        grid=(_NUM_ROW_BLOCKS, _BATCH),
        in_specs=[
            pl.BlockSpec((_ROW_BLOCK, _D_MODEL), lambda r, b: (r, 0)),
        ],
        out_specs=pl.BlockSpec((1, _ROW_BLOCK, _D_MODEL), lambda r, b: (b, r, 0)),
        out_shape=jax.ShapeDtypeStruct(
            (_BATCH, _NUM_TOKENS, _D_MODEL), jnp.float32
        ),
    )(prompt_embeddings)


# speedup vs baseline: 16.1485x; 2.2364x over previous
import jax
import jax.numpy as jnp
from jax.experimental import pallas as pl

_NUM_TOKENS = 100
_D_MODEL = 4096
_BATCH = 4


def _body(table_ref, out_ref):
    out_ref[0] = table_ref[...]


def kernel(batch_size, prompt_embeddings):
    del batch_size
    return pl.pallas_call(
        _body,
        grid=(_BATCH,),
        in_specs=[
            pl.BlockSpec((_NUM_TOKENS, _D_MODEL), lambda b: (0, 0)),
        ],
        out_specs=pl.BlockSpec((1, _NUM_TOKENS, _D_MODEL), lambda b: (b, 0, 0)),
        out_shape=jax.ShapeDtypeStruct(
            (_BATCH, _NUM_TOKENS, _D_MODEL), jnp.float32
        ),
    )(prompt_embeddings)
